# baseline (device time: 2068485 ns/iter reference)
import jax
import jax.numpy as jnp
from jax import lax
from jax.experimental import pallas as pl
from jax.experimental.pallas import tpu as pltpu

T_LOCAL = 4096
D = 2048
F = 4096
E_LOCAL = 4
TILE_F = 1024
CAP = 1152


def _gather_body(x_ref, a_ref, xall_ref, aall_ref, sems):
    my_x = lax.axis_index("x")
    my_y = lax.axis_index("y")
    peer = (my_x, 1 - my_y)

    barrier = pltpu.get_barrier_semaphore()
    pl.semaphore_signal(
        barrier, inc=1, device_id=peer, device_id_type=pl.DeviceIdType.MESH
    )
    pl.semaphore_wait(barrier, 1)

    xall_ref[pl.ds(my_y * T_LOCAL, T_LOCAL), :] = x_ref[...]
    aall_ref[pl.ds(my_y, 1), :] = a_ref[...]

    rdma_x = pltpu.make_async_remote_copy(
        src_ref=x_ref,
        dst_ref=xall_ref.at[pl.ds(my_y * T_LOCAL, T_LOCAL), :],
        send_sem=sems.at[0],
        recv_sem=sems.at[1],
        device_id=peer,
        device_id_type=pl.DeviceIdType.MESH,
    )
    rdma_a = pltpu.make_async_remote_copy(
        src_ref=a_ref,
        dst_ref=aall_ref.at[pl.ds(my_y, 1), :],
        send_sem=sems.at[2],
        recv_sem=sems.at[3],
        device_id=peer,
        device_id_type=pl.DeviceIdType.MESH,
    )
    rdma_x.start()
    rdma_a.start()
    rdma_x.wait()
    rdma_a.wait()


def _moe_body(x_ref, w1_ref, w2_ref, out_ref):
    f = pl.program_id(1)
    h = jnp.maximum(
        jnp.dot(x_ref[...], w1_ref[0], preferred_element_type=jnp.float32),
        0.0,
    ).astype(jnp.bfloat16)
    o = jnp.dot(h, w2_ref[0], preferred_element_type=jnp.float32)
    contrib = o.astype(jnp.bfloat16)

    @pl.when(f == 0)
    def _():
        out_ref[...] = contrib

    @pl.when(f != 0)
    def _():
        out_ref[...] += contrib


def _reduce_body(p_ref, out_ref, local_ref, comm_ref, sems):
    my_x = lax.axis_index("x")
    my_y = lax.axis_index("y")
    peer = (my_x, 1 - my_y)

    barrier = pltpu.get_barrier_semaphore()
    pl.semaphore_signal(
        barrier, inc=1, device_id=peer, device_id_type=pl.DeviceIdType.MESH
    )
    pl.semaphore_wait(barrier, 1)

    rdma = pltpu.make_async_remote_copy(
        src_ref=p_ref.at[pl.ds((1 - my_y) * T_LOCAL, T_LOCAL), :],
        dst_ref=comm_ref,
        send_sem=sems.at[0],
        recv_sem=sems.at[1],
        device_id=peer,
        device_id_type=pl.DeviceIdType.MESH,
    )
    rdma.start()

    local_copy = pltpu.make_async_copy(
        p_ref.at[pl.ds(my_y * T_LOCAL, T_LOCAL), :], local_ref, sems.at[2]
    )
    local_copy.start()
    local_copy.wait()
    rdma.wait()

    out_ref[...] = local_ref[...] + comm_ref[...]


def kernel(x, assign, W1, W2):
    xb = x.astype(jnp.bfloat16)
    w1b = W1.astype(jnp.bfloat16)
    w2b = W2.astype(jnp.bfloat16)
    a2 = assign.reshape(1, T_LOCAL)

    xall, aall = pl.pallas_call(
        _gather_body,
        out_shape=(
            jax.ShapeDtypeStruct((2 * T_LOCAL, D), jnp.bfloat16),
            jax.ShapeDtypeStruct((2, T_LOCAL), jnp.int32),
        ),
        in_specs=[
            pl.BlockSpec(memory_space=pltpu.VMEM),
            pl.BlockSpec(memory_space=pltpu.VMEM),
        ],
        out_specs=(
            pl.BlockSpec(memory_space=pltpu.VMEM),
            pl.BlockSpec(memory_space=pltpu.VMEM),
        ),
        scratch_shapes=[pltpu.SemaphoreType.DMA((4,))],
        compiler_params=pltpu.CompilerParams(collective_id=0),
    )(xb, a2)

    my_y = lax.axis_index("y")
    t_all = 2 * T_LOCAL
    a_flat = aall.reshape(t_all)
    order = jnp.argsort(a_flat)
    sorted_a = a_flat[order]
    starts = jnp.searchsorted(sorted_a, jnp.arange(8, dtype=jnp.int32))
    ranks = jnp.arange(t_all, dtype=jnp.int32) - starts[sorted_a]
    slot = sorted_a * CAP + ranks
    tok_of_slot = jnp.full((8 * CAP,), t_all, jnp.int32).at[slot].set(order)
    tok_mine = lax.dynamic_slice_in_dim(
        tok_of_slot, 4 * my_y * CAP, E_LOCAL * CAP
    )
    x_pad = jnp.concatenate(
        [xall, jnp.zeros((1, D), jnp.bfloat16)], axis=0
    )
    x_routed = x_pad[tok_mine]

    n_f = F // TILE_F
    y_routed = pl.pallas_call(
        _moe_body,
        grid=(E_LOCAL, n_f),
        in_specs=[
            pl.BlockSpec((CAP, D), lambda e, f: (e, 0)),
            pl.BlockSpec((1, D, TILE_F), lambda e, f: (e, 0, f)),
            pl.BlockSpec((1, TILE_F, D), lambda e, f: (e, f, 0)),
        ],
        out_specs=pl.BlockSpec((CAP, D), lambda e, f: (e, 0)),
        out_shape=jax.ShapeDtypeStruct((E_LOCAL * CAP, D), jnp.bfloat16),
        compiler_params=pltpu.CompilerParams(
            dimension_semantics=("arbitrary", "arbitrary"),
            vmem_limit_bytes=60 * 1024 * 1024,
        ),
    )(x_routed, w1b, w2b)

    partial = (
        jnp.zeros((t_all + 1, D), jnp.bfloat16)
        .at[tok_mine]
        .set(y_routed)[:t_all]
    )

    out = pl.pallas_call(
        _reduce_body,
        out_shape=jax.ShapeDtypeStruct((T_LOCAL, D), jnp.bfloat16),
        in_specs=[pl.BlockSpec(memory_space=pl.ANY)],
        out_specs=pl.BlockSpec(memory_space=pltpu.VMEM),
        scratch_shapes=[
            pltpu.VMEM((T_LOCAL, D), jnp.bfloat16),
            pltpu.VMEM((T_LOCAL, D), jnp.bfloat16),
            pltpu.SemaphoreType.DMA((3,)),
        ],
        compiler_params=pltpu.CompilerParams(collective_id=1),
    )(partial)
    return out.astype(jnp.float32)


# device time: 1273083 ns/iter; 1.6248x vs baseline; 1.6248x over previous
import jax
import jax.numpy as jnp
from jax import lax
from jax.experimental import pallas as pl
from jax.experimental.pallas import tpu as pltpu

T_LOCAL = 4096
D = 2048
F = 4096
E_LOCAL = 4
TILE_F = 1024
CAP = 1152
S_TILE = 512
K_TILE = 512


def _gather_body(x_ref, a_ref, xall_ref, aall_ref, sems):
    my_x = lax.axis_index("x")
    my_y = lax.axis_index("y")
    peer = (my_x, 1 - my_y)

    barrier = pltpu.get_barrier_semaphore()
    pl.semaphore_signal(
        barrier, inc=1, device_id=peer, device_id_type=pl.DeviceIdType.MESH
    )
    pl.semaphore_wait(barrier, 1)

    xall_ref[pl.ds(my_y * T_LOCAL, T_LOCAL), :] = x_ref[...]
    aall_ref[pl.ds(my_y, 1), :] = a_ref[...]

    rdma_x = pltpu.make_async_remote_copy(
        src_ref=x_ref,
        dst_ref=xall_ref.at[pl.ds(my_y * T_LOCAL, T_LOCAL), :],
        send_sem=sems.at[0],
        recv_sem=sems.at[1],
        device_id=peer,
        device_id_type=pl.DeviceIdType.MESH,
    )
    rdma_a = pltpu.make_async_remote_copy(
        src_ref=a_ref,
        dst_ref=aall_ref.at[pl.ds(my_y, 1), :],
        send_sem=sems.at[2],
        recv_sem=sems.at[3],
        device_id=peer,
        device_id_type=pl.DeviceIdType.MESH,
    )
    rdma_x.start()
    rdma_a.start()
    rdma_x.wait()
    rdma_a.wait()


def _gather_mm_body(ids_ref, x_ref, out_ref):
    k = pl.program_id(1)
    iota = jax.lax.broadcasted_iota(jnp.int32, (S_TILE, K_TILE), 1)
    p = (ids_ref[...] == iota + k * K_TILE).astype(jnp.bfloat16)
    contrib = jnp.dot(
        p, x_ref[...], preferred_element_type=jnp.float32
    ).astype(jnp.bfloat16)

    @pl.when(k == 0)
    def _():
        out_ref[...] = contrib

    @pl.when(k != 0)
    def _():
        out_ref[...] += contrib


def _scatter_mm_body(ids_ref, y_ref, out_ref):
    t = pl.program_id(0)
    s = pl.program_id(1)
    iota = jax.lax.broadcasted_iota(jnp.int32, (K_TILE, S_TILE), 0)
    p = (iota + t * K_TILE == ids_ref[...]).astype(jnp.bfloat16)
    contrib = jnp.dot(
        p, y_ref[...], preferred_element_type=jnp.float32
    ).astype(jnp.bfloat16)

    @pl.when(s == 0)
    def _():
        out_ref[...] = contrib

    @pl.when(s != 0)
    def _():
        out_ref[...] += contrib


def _moe_body(x_ref, w1_ref, w2_ref, out_ref):
    f = pl.program_id(1)
    h = jnp.maximum(
        jnp.dot(x_ref[...], w1_ref[0], preferred_element_type=jnp.float32),
        0.0,
    ).astype(jnp.bfloat16)
    o = jnp.dot(h, w2_ref[0], preferred_element_type=jnp.float32)
    contrib = o.astype(jnp.bfloat16)

    @pl.when(f == 0)
    def _():
        out_ref[...] = contrib

    @pl.when(f != 0)
    def _():
        out_ref[...] += contrib


def _reduce_body(p_ref, out_ref, local_ref, comm_ref, sems):
    my_x = lax.axis_index("x")
    my_y = lax.axis_index("y")
    peer = (my_x, 1 - my_y)

    barrier = pltpu.get_barrier_semaphore()
    pl.semaphore_signal(
        barrier, inc=1, device_id=peer, device_id_type=pl.DeviceIdType.MESH
    )
    pl.semaphore_wait(barrier, 1)

    rdma = pltpu.make_async_remote_copy(
        src_ref=p_ref.at[pl.ds((1 - my_y) * T_LOCAL, T_LOCAL), :],
        dst_ref=comm_ref,
        send_sem=sems.at[0],
        recv_sem=sems.at[1],
        device_id=peer,
        device_id_type=pl.DeviceIdType.MESH,
    )
    rdma.start()

    local_copy = pltpu.make_async_copy(
        p_ref.at[pl.ds(my_y * T_LOCAL, T_LOCAL), :], local_ref, sems.at[2]
    )
    local_copy.start()
    local_copy.wait()
    rdma.wait()

    out_ref[...] = local_ref[...] + comm_ref[...]


def kernel(x, assign, W1, W2):
    xb = x.astype(jnp.bfloat16)
    w1b = W1.astype(jnp.bfloat16)
    w2b = W2.astype(jnp.bfloat16)
    a2 = assign.reshape(1, T_LOCAL)

    xall, aall = pl.pallas_call(
        _gather_body,
        out_shape=(
            jax.ShapeDtypeStruct((2 * T_LOCAL, D), jnp.bfloat16),
            jax.ShapeDtypeStruct((2, T_LOCAL), jnp.int32),
        ),
        in_specs=[
            pl.BlockSpec(memory_space=pltpu.VMEM),
            pl.BlockSpec(memory_space=pltpu.VMEM),
        ],
        out_specs=(
            pl.BlockSpec(memory_space=pltpu.VMEM),
            pl.BlockSpec(memory_space=pltpu.VMEM),
        ),
        scratch_shapes=[pltpu.SemaphoreType.DMA((4,))],
        compiler_params=pltpu.CompilerParams(collective_id=0),
    )(xb, a2)

    my_y = lax.axis_index("y")
    t_all = 2 * T_LOCAL
    a_flat = aall.reshape(t_all)
    order = jnp.argsort(a_flat)
    sorted_a = a_flat[order]
    starts = jnp.searchsorted(sorted_a, jnp.arange(8, dtype=jnp.int32))
    ranks = jnp.arange(t_all, dtype=jnp.int32) - starts[sorted_a]
    slot = sorted_a * CAP + ranks
    tok_of_slot = jnp.full((8 * CAP,), t_all, jnp.int32).at[slot].set(order)
    tok_mine = lax.dynamic_slice_in_dim(
        tok_of_slot, 4 * my_y * CAP, E_LOCAL * CAP
    )

    n_slots = E_LOCAL * CAP
    x_routed = pl.pallas_call(
        _gather_mm_body,
        grid=(n_slots // S_TILE, t_all // K_TILE),
        in_specs=[
            pl.BlockSpec((S_TILE, 1), lambda s, k: (s, 0)),
            pl.BlockSpec((K_TILE, D), lambda s, k: (k, 0)),
        ],
        out_specs=pl.BlockSpec((S_TILE, D), lambda s, k: (s, 0)),
        out_shape=jax.ShapeDtypeStruct((n_slots, D), jnp.bfloat16),
        compiler_params=pltpu.CompilerParams(
            dimension_semantics=("arbitrary", "arbitrary"),
        ),
    )(tok_mine.reshape(n_slots, 1), xall)

    n_f = F // TILE_F
    y_routed = pl.pallas_call(
        _moe_body,
        grid=(E_LOCAL, n_f),
        in_specs=[
            pl.BlockSpec((CAP, D), lambda e, f: (e, 0)),
            pl.BlockSpec((1, D, TILE_F), lambda e, f: (e, 0, f)),
            pl.BlockSpec((1, TILE_F, D), lambda e, f: (e, f, 0)),
        ],
        out_specs=pl.BlockSpec((CAP, D), lambda e, f: (e, 0)),
        out_shape=jax.ShapeDtypeStruct((E_LOCAL * CAP, D), jnp.bfloat16),
        compiler_params=pltpu.CompilerParams(
            dimension_semantics=("arbitrary", "arbitrary"),
            vmem_limit_bytes=60 * 1024 * 1024,
        ),
    )(x_routed, w1b, w2b)

    partial = pl.pallas_call(
        _scatter_mm_body,
        grid=(t_all // K_TILE, n_slots // S_TILE),
        in_specs=[
            pl.BlockSpec((1, S_TILE), lambda t, s: (0, s)),
            pl.BlockSpec((S_TILE, D), lambda t, s: (s, 0)),
        ],
        out_specs=pl.BlockSpec((K_TILE, D), lambda t, s: (t, 0)),
        out_shape=jax.ShapeDtypeStruct((t_all, D), jnp.bfloat16),
        compiler_params=pltpu.CompilerParams(
            dimension_semantics=("arbitrary", "arbitrary"),
        ),
    )(tok_mine.reshape(1, n_slots), y_routed)

    out = pl.pallas_call(
        _reduce_body,
        out_shape=jax.ShapeDtypeStruct((T_LOCAL, D), jnp.bfloat16),
        in_specs=[pl.BlockSpec(memory_space=pl.ANY)],
        out_specs=pl.BlockSpec(memory_space=pltpu.VMEM),
        scratch_shapes=[
            pltpu.VMEM((T_LOCAL, D), jnp.bfloat16),
            pltpu.VMEM((T_LOCAL, D), jnp.bfloat16),
            pltpu.SemaphoreType.DMA((3,)),
        ],
        compiler_params=pltpu.CompilerParams(collective_id=1),
    )(partial)
    return out.astype(jnp.float32)


# device time: 1209313 ns/iter; 1.7105x vs baseline; 1.0527x over previous
import jax
import jax.numpy as jnp
from jax import lax
from jax.experimental import pallas as pl
from jax.experimental.pallas import tpu as pltpu

T_LOCAL = 4096
D = 2048
F = 4096
E_LOCAL = 4
TILE_F = 1024
CAP = 1152
S_TILE = 512
K_TILE = 512


def _gather_body(x_ref, a_ref, xall_ref, aall_ref, sems):
    my_x = lax.axis_index("x")
    my_y = lax.axis_index("y")
    peer = (my_x, 1 - my_y)

    barrier = pltpu.get_barrier_semaphore()
    pl.semaphore_signal(
        barrier, inc=1, device_id=peer, device_id_type=pl.DeviceIdType.MESH
    )
    pl.semaphore_wait(barrier, 1)

    xall_ref[pl.ds(my_y * T_LOCAL, T_LOCAL), :] = x_ref[...]
    aall_ref[pl.ds(my_y, 1), :] = a_ref[...]

    rdma_x = pltpu.make_async_remote_copy(
        src_ref=x_ref,
        dst_ref=xall_ref.at[pl.ds(my_y * T_LOCAL, T_LOCAL), :],
        send_sem=sems.at[0],
        recv_sem=sems.at[1],
        device_id=peer,
        device_id_type=pl.DeviceIdType.MESH,
    )
    rdma_a = pltpu.make_async_remote_copy(
        src_ref=a_ref,
        dst_ref=aall_ref.at[pl.ds(my_y, 1), :],
        send_sem=sems.at[2],
        recv_sem=sems.at[3],
        device_id=peer,
        device_id_type=pl.DeviceIdType.MESH,
    )
    rdma_x.start()
    rdma_a.start()
    rdma_x.wait()
    rdma_a.wait()


def _gather_mm_body(slot_ref, x_ref, out_ref):
    s = pl.program_id(0)
    k = pl.program_id(1)
    iota = jax.lax.broadcasted_iota(jnp.int32, (S_TILE, K_TILE), 0)
    p = (iota + s * S_TILE == slot_ref[...]).astype(jnp.bfloat16)
    contrib = jnp.dot(
        p, x_ref[...], preferred_element_type=jnp.float32
    ).astype(jnp.bfloat16)

    @pl.when(k == 0)
    def _():
        out_ref[...] = contrib

    @pl.when(k != 0)
    def _():
        out_ref[...] += contrib


def _scatter_mm_body(slot_ref, y_ref, out_ref):
    s = pl.program_id(1)
    iota = jax.lax.broadcasted_iota(jnp.int32, (K_TILE, S_TILE), 1)
    p = (slot_ref[...] == iota + s * S_TILE).astype(jnp.bfloat16)
    contrib = jnp.dot(
        p, y_ref[...], preferred_element_type=jnp.float32
    ).astype(jnp.bfloat16)

    @pl.when(s == 0)
    def _():
        out_ref[...] = contrib

    @pl.when(s != 0)
    def _():
        out_ref[...] += contrib


def _moe_body(x_ref, w1_ref, w2_ref, out_ref):
    f = pl.program_id(1)
    h = jnp.maximum(
        jnp.dot(x_ref[...], w1_ref[0], preferred_element_type=jnp.float32),
        0.0,
    ).astype(jnp.bfloat16)
    o = jnp.dot(h, w2_ref[0], preferred_element_type=jnp.float32)
    contrib = o.astype(jnp.bfloat16)

    @pl.when(f == 0)
    def _():
        out_ref[...] = contrib

    @pl.when(f != 0)
    def _():
        out_ref[...] += contrib


def _reduce_body(p_ref, out_ref, local_ref, comm_ref, sems):
    my_x = lax.axis_index("x")
    my_y = lax.axis_index("y")
    peer = (my_x, 1 - my_y)

    barrier = pltpu.get_barrier_semaphore()
    pl.semaphore_signal(
        barrier, inc=1, device_id=peer, device_id_type=pl.DeviceIdType.MESH
    )
    pl.semaphore_wait(barrier, 1)

    rdma = pltpu.make_async_remote_copy(
        src_ref=p_ref.at[pl.ds((1 - my_y) * T_LOCAL, T_LOCAL), :],
        dst_ref=comm_ref,
        send_sem=sems.at[0],
        recv_sem=sems.at[1],
        device_id=peer,
        device_id_type=pl.DeviceIdType.MESH,
    )
    rdma.start()

    local_copy = pltpu.make_async_copy(
        p_ref.at[pl.ds(my_y * T_LOCAL, T_LOCAL), :], local_ref, sems.at[2]
    )
    local_copy.start()
    local_copy.wait()
    rdma.wait()

    out_ref[...] = local_ref[...] + comm_ref[...]


def kernel(x, assign, W1, W2):
    xb = x.astype(jnp.bfloat16)
    w1b = W1.astype(jnp.bfloat16)
    w2b = W2.astype(jnp.bfloat16)
    a2 = assign.reshape(1, T_LOCAL)

    xall, aall = pl.pallas_call(
        _gather_body,
        out_shape=(
            jax.ShapeDtypeStruct((2 * T_LOCAL, D), jnp.bfloat16),
            jax.ShapeDtypeStruct((2, T_LOCAL), jnp.int32),
        ),
        in_specs=[
            pl.BlockSpec(memory_space=pltpu.VMEM),
            pl.BlockSpec(memory_space=pltpu.VMEM),
        ],
        out_specs=(
            pl.BlockSpec(memory_space=pltpu.VMEM),
            pl.BlockSpec(memory_space=pltpu.VMEM),
        ),
        scratch_shapes=[pltpu.SemaphoreType.DMA((4,))],
        compiler_params=pltpu.CompilerParams(collective_id=0),
    )(xb, a2)

    my_y = lax.axis_index("y")
    t_all = 2 * T_LOCAL
    a_flat = aall.reshape(t_all)
    onehot = (
        a_flat[:, None] == jnp.arange(8, dtype=jnp.int32)[None, :]
    ).astype(jnp.int32)
    rank = jnp.sum((jnp.cumsum(onehot, axis=0) - onehot) * onehot, axis=1)
    slot_tok = a_flat * CAP + rank - 4 * my_y * CAP

    n_slots = E_LOCAL * CAP
    x_routed = pl.pallas_call(
        _gather_mm_body,
        grid=(n_slots // S_TILE, t_all // K_TILE),
        in_specs=[
            pl.BlockSpec((1, K_TILE), lambda s, k: (0, k)),
            pl.BlockSpec((K_TILE, D), lambda s, k: (k, 0)),
        ],
        out_specs=pl.BlockSpec((S_TILE, D), lambda s, k: (s, 0)),
        out_shape=jax.ShapeDtypeStruct((n_slots, D), jnp.bfloat16),
        compiler_params=pltpu.CompilerParams(
            dimension_semantics=("arbitrary", "arbitrary"),
        ),
    )(slot_tok.reshape(1, t_all), xall)

    n_f = F // TILE_F
    y_routed = pl.pallas_call(
        _moe_body,
        grid=(E_LOCAL, n_f),
        in_specs=[
            pl.BlockSpec((CAP, D), lambda e, f: (e, 0)),
            pl.BlockSpec((1, D, TILE_F), lambda e, f: (e, 0, f)),
            pl.BlockSpec((1, TILE_F, D), lambda e, f: (e, f, 0)),
        ],
        out_specs=pl.BlockSpec((CAP, D), lambda e, f: (e, 0)),
        out_shape=jax.ShapeDtypeStruct((E_LOCAL * CAP, D), jnp.bfloat16),
        compiler_params=pltpu.CompilerParams(
            dimension_semantics=("arbitrary", "arbitrary"),
            vmem_limit_bytes=60 * 1024 * 1024,
        ),
    )(x_routed, w1b, w2b)

    partial = pl.pallas_call(
        _scatter_mm_body,
        grid=(t_all // K_TILE, n_slots // S_TILE),
        in_specs=[
            pl.BlockSpec((K_TILE, 1), lambda t, s: (t, 0)),
            pl.BlockSpec((S_TILE, D), lambda t, s: (s, 0)),
        ],
        out_specs=pl.BlockSpec((K_TILE, D), lambda t, s: (t, 0)),
        out_shape=jax.ShapeDtypeStruct((t_all, D), jnp.bfloat16),
        compiler_params=pltpu.CompilerParams(
            dimension_semantics=("arbitrary", "arbitrary"),
        ),
    )(slot_tok.reshape(t_all, 1), y_routed)

    out = pl.pallas_call(
        _reduce_body,
        out_shape=jax.ShapeDtypeStruct((T_LOCAL, D), jnp.bfloat16),
        in_specs=[pl.BlockSpec(memory_space=pl.ANY)],
        out_specs=pl.BlockSpec(memory_space=pltpu.VMEM),
        scratch_shapes=[
            pltpu.VMEM((T_LOCAL, D), jnp.bfloat16),
            pltpu.VMEM((T_LOCAL, D), jnp.bfloat16),
            pltpu.SemaphoreType.DMA((3,)),
        ],
        compiler_params=pltpu.CompilerParams(collective_id=1),
    )(partial)
    return out.astype(jnp.float32)


# device time: 1094895 ns/iter; 1.8892x vs baseline; 1.1045x over previous
import jax
import jax.numpy as jnp
from jax import lax
from jax.experimental import pallas as pl
from jax.experimental.pallas import tpu as pltpu

T_LOCAL = 4096
D = 2048
F = 4096
E_LOCAL = 4
TILE_F = 512
CAP = 1152
S_TILE = 512
K_TILE = 512


def _gather_body(x_ref, a_ref, xall_ref, aall_ref, sems):
    my_x = lax.axis_index("x")
    my_y = lax.axis_index("y")
    peer = (my_x, 1 - my_y)

    barrier = pltpu.get_barrier_semaphore()
    pl.semaphore_signal(
        barrier, inc=1, device_id=peer, device_id_type=pl.DeviceIdType.MESH
    )
    pl.semaphore_wait(barrier, 1)

    xall_ref[pl.ds(my_y * T_LOCAL, T_LOCAL), :] = x_ref[...]
    aall_ref[pl.ds(my_y, 1), :] = a_ref[...]

    rdma_x = pltpu.make_async_remote_copy(
        src_ref=x_ref,
        dst_ref=xall_ref.at[pl.ds(my_y * T_LOCAL, T_LOCAL), :],
        send_sem=sems.at[0],
        recv_sem=sems.at[1],
        device_id=peer,
        device_id_type=pl.DeviceIdType.MESH,
    )
    rdma_a = pltpu.make_async_remote_copy(
        src_ref=a_ref,
        dst_ref=aall_ref.at[pl.ds(my_y, 1), :],
        send_sem=sems.at[2],
        recv_sem=sems.at[3],
        device_id=peer,
        device_id_type=pl.DeviceIdType.MESH,
    )
    rdma_x.start()
    rdma_a.start()
    rdma_x.wait()
    rdma_a.wait()


def _gather_mm_body(slot_ref, x_ref, out_ref):
    s = pl.program_id(0)
    k = pl.program_id(1)
    iota = jax.lax.broadcasted_iota(jnp.int32, (S_TILE, K_TILE), 0)
    p = (iota + s * S_TILE == slot_ref[...]).astype(jnp.bfloat16)
    contrib = jnp.dot(
        p, x_ref[...], preferred_element_type=jnp.float32
    ).astype(jnp.bfloat16)

    @pl.when(k == 0)
    def _():
        out_ref[...] = contrib

    @pl.when(k != 0)
    def _():
        out_ref[...] += contrib


def _scatter_mm_body(slot_ref, y_ref, out_ref):
    s = pl.program_id(1)
    iota = jax.lax.broadcasted_iota(jnp.int32, (K_TILE, S_TILE), 1)
    p = (slot_ref[...] == iota + s * S_TILE).astype(jnp.bfloat16)
    contrib = jnp.dot(
        p, y_ref[...], preferred_element_type=jnp.float32
    ).astype(jnp.bfloat16)

    @pl.when(s == 0)
    def _():
        out_ref[...] = contrib

    @pl.when(s != 0)
    def _():
        out_ref[...] += contrib


def _moe_body(x_ref, w1_ref, w2_ref, out_ref):
    f = pl.program_id(1)
    w1 = w1_ref[0].astype(jnp.bfloat16)
    w2 = w2_ref[0].astype(jnp.bfloat16)
    h = jnp.maximum(
        jnp.dot(x_ref[...], w1, preferred_element_type=jnp.float32),
        0.0,
    ).astype(jnp.bfloat16)
    o = jnp.dot(h, w2, preferred_element_type=jnp.float32)
    contrib = o.astype(jnp.bfloat16)

    @pl.when(f == 0)
    def _():
        out_ref[...] = contrib

    @pl.when(f != 0)
    def _():
        out_ref[...] += contrib


def _reduce_body(p_ref, out_ref, local_ref, comm_ref, sems):
    my_x = lax.axis_index("x")
    my_y = lax.axis_index("y")
    peer = (my_x, 1 - my_y)

    barrier = pltpu.get_barrier_semaphore()
    pl.semaphore_signal(
        barrier, inc=1, device_id=peer, device_id_type=pl.DeviceIdType.MESH
    )
    pl.semaphore_wait(barrier, 1)

    rdma = pltpu.make_async_remote_copy(
        src_ref=p_ref.at[pl.ds((1 - my_y) * T_LOCAL, T_LOCAL), :],
        dst_ref=comm_ref,
        send_sem=sems.at[0],
        recv_sem=sems.at[1],
        device_id=peer,
        device_id_type=pl.DeviceIdType.MESH,
    )
    rdma.start()

    local_copy = pltpu.make_async_copy(
        p_ref.at[pl.ds(my_y * T_LOCAL, T_LOCAL), :], local_ref, sems.at[2]
    )
    local_copy.start()
    local_copy.wait()
    rdma.wait()

    out_ref[...] = local_ref[...] + comm_ref[...]


def kernel(x, assign, W1, W2):
    xb = x.astype(jnp.bfloat16)
    a2 = assign.reshape(1, T_LOCAL)

    xall, aall = pl.pallas_call(
        _gather_body,
        out_shape=(
            jax.ShapeDtypeStruct((2 * T_LOCAL, D), jnp.bfloat16),
            jax.ShapeDtypeStruct((2, T_LOCAL), jnp.int32),
        ),
        in_specs=[
            pl.BlockSpec(memory_space=pltpu.VMEM),
            pl.BlockSpec(memory_space=pltpu.VMEM),
        ],
        out_specs=(
            pl.BlockSpec(memory_space=pltpu.VMEM),
            pl.BlockSpec(memory_space=pltpu.VMEM),
        ),
        scratch_shapes=[pltpu.SemaphoreType.DMA((4,))],
        compiler_params=pltpu.CompilerParams(collective_id=0),
    )(xb, a2)

    my_y = lax.axis_index("y")
    t_all = 2 * T_LOCAL
    a_flat = aall.reshape(t_all)
    onehot = (
        a_flat[:, None] == jnp.arange(8, dtype=jnp.int32)[None, :]
    ).astype(jnp.int32)
    rank = jnp.sum((jnp.cumsum(onehot, axis=0) - onehot) * onehot, axis=1)
    slot_tok = a_flat * CAP + rank - 4 * my_y * CAP

    n_slots = E_LOCAL * CAP
    x_routed = pl.pallas_call(
        _gather_mm_body,
        grid=(n_slots // S_TILE, t_all // K_TILE),
        in_specs=[
            pl.BlockSpec((1, K_TILE), lambda s, k: (0, k)),
            pl.BlockSpec((K_TILE, D), lambda s, k: (k, 0)),
        ],
        out_specs=pl.BlockSpec((S_TILE, D), lambda s, k: (s, 0)),
        out_shape=jax.ShapeDtypeStruct((n_slots, D), jnp.bfloat16),
        compiler_params=pltpu.CompilerParams(
            dimension_semantics=("arbitrary", "arbitrary"),
        ),
    )(slot_tok.reshape(1, t_all), xall)

    n_f = F // TILE_F
    y_routed = pl.pallas_call(
        _moe_body,
        grid=(E_LOCAL, n_f),
        in_specs=[
            pl.BlockSpec((CAP, D), lambda e, f: (e, 0)),
            pl.BlockSpec((1, D, TILE_F), lambda e, f: (e, 0, f)),
            pl.BlockSpec((1, TILE_F, D), lambda e, f: (e, f, 0)),
        ],
        out_specs=pl.BlockSpec((CAP, D), lambda e, f: (e, 0)),
        out_shape=jax.ShapeDtypeStruct((E_LOCAL * CAP, D), jnp.bfloat16),
        compiler_params=pltpu.CompilerParams(
            dimension_semantics=("arbitrary", "arbitrary"),
            vmem_limit_bytes=60 * 1024 * 1024,
        ),
    )(x_routed, W1, W2)

    partial = pl.pallas_call(
        _scatter_mm_body,
        grid=(t_all // K_TILE, n_slots // S_TILE),
        in_specs=[
            pl.BlockSpec((K_TILE, 1), lambda t, s: (t, 0)),
            pl.BlockSpec((S_TILE, D), lambda t, s: (s, 0)),
        ],
        out_specs=pl.BlockSpec((K_TILE, D), lambda t, s: (t, 0)),
        out_shape=jax.ShapeDtypeStruct((t_all, D), jnp.bfloat16),
        compiler_params=pltpu.CompilerParams(
            dimension_semantics=("arbitrary", "arbitrary"),
        ),
    )(slot_tok.reshape(t_all, 1), y_routed)

    out = pl.pallas_call(
        _reduce_body,
        out_shape=jax.ShapeDtypeStruct((T_LOCAL, D), jnp.bfloat16),
        in_specs=[pl.BlockSpec(memory_space=pl.ANY)],
        out_specs=pl.BlockSpec(memory_space=pltpu.VMEM),
        scratch_shapes=[
            pltpu.VMEM((T_LOCAL, D), jnp.bfloat16),
            pltpu.VMEM((T_LOCAL, D), jnp.bfloat16),
            pltpu.SemaphoreType.DMA((3,)),
        ],
        compiler_params=pltpu.CompilerParams(collective_id=1),
    )(partial)
    return out.astype(jnp.float32)


# device time: 956395 ns/iter; 2.1628x vs baseline; 1.1448x over previous
import jax
import jax.numpy as jnp
from jax import lax
from jax.experimental import pallas as pl
from jax.experimental.pallas import tpu as pltpu

T_LOCAL = 4096
D = 2048
F = 4096
E_LOCAL = 4
TILE_F = 512
CAP = 1152
S_TILE = 512
K_TILE = 512
N_SLOTS = E_LOCAL * CAP


def _assign_xchg_body(a_ref, aall_ref, sems):
    my_x = lax.axis_index("x")
    my_y = lax.axis_index("y")
    peer = (my_x, 1 - my_y)

    barrier = pltpu.get_barrier_semaphore()
    pl.semaphore_signal(
        barrier, inc=1, device_id=peer, device_id_type=pl.DeviceIdType.MESH
    )
    pl.semaphore_wait(barrier, 1)

    aall_ref[pl.ds(my_y, 1), :] = a_ref[...]
    rdma = pltpu.make_async_remote_copy(
        src_ref=a_ref,
        dst_ref=aall_ref.at[pl.ds(my_y, 1), :],
        send_sem=sems.at[0],
        recv_sem=sems.at[1],
        device_id=peer,
        device_id_type=pl.DeviceIdType.MESH,
    )
    rdma.start()
    rdma.wait()


def _xchg_gather_body(x_ref, slot_ref, out_ref, comm_ref, sems):
    my_x = lax.axis_index("x")
    my_y = lax.axis_index("y")
    peer = (my_x, 1 - my_y)

    barrier = pltpu.get_barrier_semaphore()
    pl.semaphore_signal(
        barrier, inc=1, device_id=peer, device_id_type=pl.DeviceIdType.MESH
    )
    pl.semaphore_wait(barrier, 1)

    rdma = pltpu.make_async_remote_copy(
        src_ref=x_ref,
        dst_ref=comm_ref,
        send_sem=sems.at[0],
        recv_sem=sems.at[1],
        device_id=peer,
        device_id_type=pl.DeviceIdType.MESH,
    )
    rdma.start()

    out_ref[...] = jnp.zeros_like(out_ref)

    def gather_from(buf_ref, tok_base):
        for kk in range(T_LOCAL // K_TILE):
            ids = slot_ref[:, pl.ds(tok_base + kk * K_TILE, K_TILE)]
            xblk = buf_ref[kk * K_TILE : (kk + 1) * K_TILE, :]
            for s in range(N_SLOTS // S_TILE):
                iota = jax.lax.broadcasted_iota(
                    jnp.int32, (S_TILE, K_TILE), 0
                )
                p = (iota + s * S_TILE == ids).astype(jnp.bfloat16)
                contrib = jnp.dot(
                    p, xblk, preferred_element_type=jnp.float32
                ).astype(jnp.bfloat16)
                out_ref[s * S_TILE : (s + 1) * S_TILE, :] += contrib

    gather_from(x_ref, my_y * T_LOCAL)
    rdma.wait()
    gather_from(comm_ref, (1 - my_y) * T_LOCAL)


def _scatter_mm_body(slot_ref, y_ref, out_ref):
    s = pl.program_id(1)
    iota = jax.lax.broadcasted_iota(jnp.int32, (K_TILE, S_TILE), 1)
    p = (slot_ref[...] == iota + s * S_TILE).astype(jnp.bfloat16)
    contrib = jnp.dot(
        p, y_ref[...], preferred_element_type=jnp.float32
    ).astype(jnp.bfloat16)

    @pl.when(s == 0)
    def _():
        out_ref[...] = contrib

    @pl.when(s != 0)
    def _():
        out_ref[...] += contrib


def _moe_body(x_ref, w1_ref, w2_ref, out_ref):
    f = pl.program_id(1)
    w1 = w1_ref[0].astype(jnp.bfloat16)
    w2 = w2_ref[0].astype(jnp.bfloat16)
    h = jnp.maximum(
        jnp.dot(x_ref[...], w1, preferred_element_type=jnp.float32),
        0.0,
    ).astype(jnp.bfloat16)
    o = jnp.dot(h, w2, preferred_element_type=jnp.float32)
    contrib = o.astype(jnp.bfloat16)

    @pl.when(f == 0)
    def _():
        out_ref[...] = contrib

    @pl.when(f != 0)
    def _():
        out_ref[...] += contrib


def _reduce_body(p_ref, out_ref, local_ref, comm_ref, sems):
    my_x = lax.axis_index("x")
    my_y = lax.axis_index("y")
    peer = (my_x, 1 - my_y)

    barrier = pltpu.get_barrier_semaphore()
    pl.semaphore_signal(
        barrier, inc=1, device_id=peer, device_id_type=pl.DeviceIdType.MESH
    )
    pl.semaphore_wait(barrier, 1)

    rdma = pltpu.make_async_remote_copy(
        src_ref=p_ref.at[pl.ds((1 - my_y) * T_LOCAL, T_LOCAL), :],
        dst_ref=comm_ref,
        send_sem=sems.at[0],
        recv_sem=sems.at[1],
        device_id=peer,
        device_id_type=pl.DeviceIdType.MESH,
    )
    rdma.start()

    local_copy = pltpu.make_async_copy(
        p_ref.at[pl.ds(my_y * T_LOCAL, T_LOCAL), :], local_ref, sems.at[2]
    )
    local_copy.start()
    local_copy.wait()
    rdma.wait()

    out_ref[...] = local_ref[...] + comm_ref[...]


def kernel(x, assign, W1, W2):
    xb = x.astype(jnp.bfloat16)
    a2 = assign.reshape(1, T_LOCAL)

    aall = pl.pallas_call(
        _assign_xchg_body,
        out_shape=jax.ShapeDtypeStruct((2, T_LOCAL), jnp.int32),
        in_specs=[pl.BlockSpec(memory_space=pltpu.VMEM)],
        out_specs=pl.BlockSpec(memory_space=pltpu.VMEM),
        scratch_shapes=[pltpu.SemaphoreType.DMA((2,))],
        compiler_params=pltpu.CompilerParams(collective_id=0),
    )(a2)

    my_y = lax.axis_index("y")
    t_all = 2 * T_LOCAL
    a_flat = aall.reshape(t_all)
    onehot = (
        a_flat[:, None] == jnp.arange(8, dtype=jnp.int32)[None, :]
    ).astype(jnp.int32)
    rank = jnp.sum((jnp.cumsum(onehot, axis=0) - onehot) * onehot, axis=1)
    slot_tok = a_flat * CAP + rank - 4 * my_y * CAP

    n_slots = N_SLOTS
    x_routed = pl.pallas_call(
        _xchg_gather_body,
        out_shape=jax.ShapeDtypeStruct((n_slots, D), jnp.bfloat16),
        in_specs=[
            pl.BlockSpec(memory_space=pltpu.VMEM),
            pl.BlockSpec(memory_space=pltpu.VMEM),
        ],
        out_specs=pl.BlockSpec(memory_space=pltpu.VMEM),
        scratch_shapes=[
            pltpu.VMEM((T_LOCAL, D), jnp.bfloat16),
            pltpu.SemaphoreType.DMA((2,)),
        ],
        compiler_params=pltpu.CompilerParams(
            collective_id=1,
            vmem_limit_bytes=60 * 1024 * 1024,
        ),
    )(xb, slot_tok.reshape(1, t_all))

    n_f = F // TILE_F
    y_routed = pl.pallas_call(
        _moe_body,
        grid=(E_LOCAL, n_f),
        in_specs=[
            pl.BlockSpec((CAP, D), lambda e, f: (e, 0)),
            pl.BlockSpec((1, D, TILE_F), lambda e, f: (e, 0, f)),
            pl.BlockSpec((1, TILE_F, D), lambda e, f: (e, f, 0)),
        ],
        out_specs=pl.BlockSpec((CAP, D), lambda e, f: (e, 0)),
        out_shape=jax.ShapeDtypeStruct((E_LOCAL * CAP, D), jnp.bfloat16),
        compiler_params=pltpu.CompilerParams(
            dimension_semantics=("arbitrary", "arbitrary"),
            vmem_limit_bytes=60 * 1024 * 1024,
        ),
    )(x_routed, W1, W2)

    partial = pl.pallas_call(
        _scatter_mm_body,
        grid=(t_all // K_TILE, n_slots // S_TILE),
        in_specs=[
            pl.BlockSpec((K_TILE, 1), lambda t, s: (t, 0)),
            pl.BlockSpec((S_TILE, D), lambda t, s: (s, 0)),
        ],
        out_specs=pl.BlockSpec((K_TILE, D), lambda t, s: (t, 0)),
        out_shape=jax.ShapeDtypeStruct((t_all, D), jnp.bfloat16),
        compiler_params=pltpu.CompilerParams(
            dimension_semantics=("arbitrary", "arbitrary"),
        ),
    )(slot_tok.reshape(t_all, 1), y_routed)

    out = pl.pallas_call(
        _reduce_body,
        out_shape=jax.ShapeDtypeStruct((T_LOCAL, D), jnp.bfloat16),
        in_specs=[pl.BlockSpec(memory_space=pl.ANY)],
        out_specs=pl.BlockSpec(memory_space=pltpu.VMEM),
        scratch_shapes=[
            pltpu.VMEM((T_LOCAL, D), jnp.bfloat16),
            pltpu.VMEM((T_LOCAL, D), jnp.bfloat16),
            pltpu.SemaphoreType.DMA((3,)),
        ],
        compiler_params=pltpu.CompilerParams(collective_id=2),
    )(partial)
    return out.astype(jnp.float32)
